# Initial kernel scaffold; baseline (speedup 1.0000x reference)
import jax
import jax.numpy as jnp
from jax.experimental import pallas as pl


def kernel(x, w_gate, W1, b1, W2, b2):
    raise NotImplementedError("write your pallas kernel here")


# speedup vs baseline: 2.3659x; 2.3659x over previous
import functools
import math

import jax
import jax.numpy as jnp
from jax import lax
from jax.experimental import pallas as pl
from jax.experimental.pallas import tpu as pltpu

E = 8
K = 2
TILE = 128

_SQRT2 = math.sqrt(2.0)


def _gelu_exact(v):
    return 0.5 * v * (1.0 + lax.erf(v / _SQRT2))


def _top2_gates(logits, e):
    iota = lax.broadcasted_iota(jnp.int32, logits.shape, 1)
    m1 = jnp.max(logits, axis=1, keepdims=True)
    i1 = jnp.min(jnp.where(logits == m1, iota, E), axis=1, keepdims=True)
    masked = jnp.where(iota == i1, -jnp.inf, logits)
    m2 = jnp.max(masked, axis=1, keepdims=True)
    i2 = jnp.min(jnp.where(masked == m2, iota, E), axis=1, keepdims=True)
    a = jnp.exp(m2 - m1)
    g1 = 1.0 / (1.0 + a)
    g2 = a / (1.0 + a)
    return jnp.where(i1 == e, g1, 0.0) + jnp.where(i2 == e, g2, 0.0)


def _dense_body(x_ref, wg_ref, W1_ref, b1_ref, W2_ref, b2_ref, out_ref):
    e = pl.program_id(0)
    t = pl.program_id(1)

    @pl.when((e == 0) & (t == 0))
    def _init():
        out_ref[...] = jnp.zeros_like(out_ref)

    xb = x_ref[...]
    logits = jnp.dot(xb, wg_ref[...], preferred_element_type=jnp.float32)
    ge = _top2_gates(logits, e)

    h = _gelu_exact(
        jnp.dot(xb, W1_ref[0], preferred_element_type=jnp.float32) + b1_ref[0]
    )
    out = jnp.dot(h, W2_ref[0], preferred_element_type=jnp.float32) + b2_ref[0]
    out_ref[pl.ds(t * TILE, TILE), :] += ge * out


@jax.jit
def kernel(x, w_gate, W1, b1, W2, b2):
    N, D = x.shape
    H = W1.shape[2]
    n_tiles = N // TILE
    return pl.pallas_call(
        _dense_body,
        grid=(E, n_tiles),
        in_specs=[
            pl.BlockSpec((TILE, D), lambda e, t: (t, 0)),
            pl.BlockSpec((D, E), lambda e, t: (0, 0)),
            pl.BlockSpec((1, D, H), lambda e, t: (e, 0, 0)),
            pl.BlockSpec((1, 1, H), lambda e, t: (e, 0, 0)),
            pl.BlockSpec((1, H, D), lambda e, t: (e, 0, 0)),
            pl.BlockSpec((1, 1, D), lambda e, t: (e, 0, 0)),
        ],
        out_specs=pl.BlockSpec((N, D), lambda e, t: (0, 0)),
        out_shape=jax.ShapeDtypeStruct((N, D), jnp.float32),
        compiler_params=pltpu.CompilerParams(
            dimension_semantics=("arbitrary", "arbitrary"),
        ),
    )(x, w_gate, W1, b1.reshape(E, 1, H), W2, b2.reshape(E, 1, D))


# speedup vs baseline: 3.0942x; 1.3078x over previous
import functools
import math

import jax
import jax.numpy as jnp
from jax import lax
from jax.experimental import pallas as pl
from jax.experimental.pallas import tpu as pltpu
from jax.experimental.pallas import tpu_sc as plsc

E = 8
K = 2
N = 2048
D = 768
H = 3072

TILE = 128
P = 5120
N_TILES = P // TILE
NC, NS = 2, 16
NW = NC * NS
TOK_W = N // NW
CHUNK = 32
NCH = TOK_W // CHUNK

_SQRT2 = math.sqrt(2.0)


def _gelu_exact(v):
    return 0.5 * v * (1.0 + lax.erf(v / _SQRT2))


def _gating_body(x_ref, wg_ref, i1_ref, i2_ref, g1_ref, g2_ref,
                 r1_ref, r2_ref, cnt_ref):
    logits = jnp.dot(x_ref[...], wg_ref[...], preferred_element_type=jnp.float32)
    iota = lax.broadcasted_iota(jnp.int32, logits.shape, 1)
    m1 = jnp.max(logits, axis=1, keepdims=True)
    i1 = jnp.min(jnp.where(logits == m1, iota, E), axis=1, keepdims=True)
    masked = jnp.where(iota == i1, -jnp.inf, logits)
    m2 = jnp.max(masked, axis=1, keepdims=True)
    i2 = jnp.min(jnp.where(masked == m2, iota, E), axis=1, keepdims=True)
    a = jnp.exp(m2 - m1)
    lanes16 = jnp.ones((1, 16), jnp.float32)
    g1_ref[...] = (1.0 / (1.0 + a)) * lanes16
    g2_ref[...] = (a / (1.0 + a)) * lanes16
    i1_ref[...] = i1
    i2_ref[...] = i2

    ri = lax.broadcasted_iota(jnp.int32, (TILE, TILE), 0)
    ci = lax.broadcasted_iota(jnp.int32, (TILE, TILE), 1)
    l_strict = (ri > ci).astype(jnp.float32)
    e_iota = lax.broadcasted_iota(jnp.int32, (TILE, E), 1)

    def block(b, idx_ref, r_ref, carry):
        idx = idx_ref[pl.ds(b * TILE, TILE), :]
        oh = (idx == e_iota).astype(jnp.float32)
        excl = jnp.dot(l_strict, oh, preferred_element_type=jnp.float32) + carry
        r_ref[pl.ds(b * TILE, TILE), :] = (
            jnp.sum(excl * oh, axis=1, keepdims=True).astype(jnp.int32))
        return carry + jnp.sum(oh, axis=0, keepdims=True)

    carry = jnp.zeros((1, E), jnp.float32)
    carry = lax.fori_loop(0, N // TILE, lambda b, c: block(b, i1_ref, r1_ref, c),
                          carry)
    carry = lax.fori_loop(0, N // TILE, lambda b, c: block(b, i2_ref, r2_ref, c),
                          carry)
    cnt_ref[...] = carry.astype(jnp.int32)


def _gating(x, w_gate):
    sds = jax.ShapeDtypeStruct
    return pl.pallas_call(
        _gating_body,
        out_shape=(
            sds((N, 1), jnp.int32), sds((N, 1), jnp.int32),
            sds((N, 16), jnp.float32), sds((N, 16), jnp.float32),
            sds((N, 1), jnp.int32), sds((N, 1), jnp.int32),
            sds((1, E), jnp.int32),
        ),
    )(x, w_gate)


def _dispatch_body(x_hbm, sp_hbm, xs_hbm, xbuf, idx_v):
    wid = lax.axis_index("s") * NC + lax.axis_index("c")
    pltpu.sync_copy(sp_hbm.at[wid], idx_v)
    for ch in range(NCH):
        pltpu.sync_copy(x_hbm.at[pl.ds(wid * TOK_W + ch * CHUNK, CHUNK)], xbuf)
        pltpu.sync_copy(xbuf, xs_hbm.at[idx_v.at[ch]])
        pltpu.sync_copy(xbuf, xs_hbm.at[idx_v.at[ch + NCH]])


def _dispatch(x, sp_all):
    mesh = plsc.VectorSubcoreMesh(core_axis_name="c", subcore_axis_name="s",
                                  num_cores=NC, num_subcores=NS)
    return pl.kernel(
        _dispatch_body,
        out_type=jax.ShapeDtypeStruct((P, D), jnp.float32),
        mesh=mesh,
        scratch_types=[
            pltpu.VMEM((CHUNK, D), jnp.float32),
            pltpu.VMEM((2 * NCH, CHUNK), jnp.int32),
        ],
    )(x, sp_all)


def _mlp_body(eid_ref, xs_ref, W1_ref, b1_ref, W2_ref, b2_ref, out_ref):
    del eid_ref
    h = _gelu_exact(
        jnp.dot(xs_ref[...], W1_ref[0], preferred_element_type=jnp.float32)
        + b1_ref[0])
    out_ref[...] = (
        jnp.dot(h, W2_ref[0], preferred_element_type=jnp.float32) + b2_ref[0])


def _grouped_mlp(tile_eid, x_sorted, W1, b1, W2, b2):
    grid_spec = pltpu.PrefetchScalarGridSpec(
        num_scalar_prefetch=1,
        grid=(N_TILES,),
        in_specs=[
            pl.BlockSpec((TILE, D), lambda i, eid: (i, 0)),
            pl.BlockSpec((1, D, H), lambda i, eid: (eid[i], 0, 0)),
            pl.BlockSpec((1, 1, H), lambda i, eid: (eid[i], 0, 0)),
            pl.BlockSpec((1, H, D), lambda i, eid: (eid[i], 0, 0)),
            pl.BlockSpec((1, 1, D), lambda i, eid: (eid[i], 0, 0)),
        ],
        out_specs=pl.BlockSpec((TILE, D), lambda i, eid: (i, 0)),
    )
    return pl.pallas_call(
        _mlp_body,
        grid_spec=grid_spec,
        out_shape=jax.ShapeDtypeStruct((P, D), jnp.float32),
        compiler_params=pltpu.CompilerParams(
            dimension_semantics=("arbitrary",),
        ),
    )(tile_eid, x_sorted, W1, b1.reshape(E, 1, H), W2, b2.reshape(E, 1, D))


def _combine_body(out_hbm, sp_hbm, g_hbm, y_hbm, r1b, r2b, yb, idx_v, g_v):
    wid = lax.axis_index("s") * NC + lax.axis_index("c")
    pltpu.sync_copy(sp_hbm.at[wid], idx_v)
    pltpu.sync_copy(g_hbm.at[wid], g_v)
    for ch in range(NCH):
        pltpu.sync_copy(out_hbm.at[idx_v.at[ch]], r1b)
        pltpu.sync_copy(out_hbm.at[idx_v.at[ch + NCH]], r2b)

        def token(j, _):
            ga = g_v[ch * CHUNK + j, :]
            gb = g_v[(ch + NCH) * CHUNK + j, :]
            for c in range(D // 16):
                a = r1b[j, pl.ds(c * 16, 16)]
                b = r2b[j, pl.ds(c * 16, 16)]
                yb[j, pl.ds(c * 16, 16)] = ga * a + gb * b
            return 0

        lax.fori_loop(0, CHUNK, token, 0)
        pltpu.sync_copy(yb, y_hbm.at[pl.ds(wid * TOK_W + ch * CHUNK, CHUNK)])


def _combine(out_all, sp_all, g_all):
    mesh = plsc.VectorSubcoreMesh(core_axis_name="c", subcore_axis_name="s",
                                  num_cores=NC, num_subcores=NS)
    return pl.kernel(
        _combine_body,
        out_type=jax.ShapeDtypeStruct((N, D), jnp.float32),
        mesh=mesh,
        scratch_types=[
            pltpu.VMEM((CHUNK, D), jnp.float32),
            pltpu.VMEM((CHUNK, D), jnp.float32),
            pltpu.VMEM((CHUNK, D), jnp.float32),
            pltpu.VMEM((2 * NCH, CHUNK), jnp.int32),
            pltpu.VMEM((2 * NCH * CHUNK, 16), jnp.float32),
        ],
    )(out_all, sp_all, g_all)


@jax.jit
def kernel(x, w_gate, W1, b1, W2, b2):
    i1, i2, g1, g2, r1, r2, cnt = _gating(x, w_gate)
    i1, i2 = i1[:, 0], i2[:, 0]
    r1, r2 = r1[:, 0], r2[:, 0]
    counts = cnt[0]

    padded = ((counts + TILE - 1) // TILE) * TILE
    ends = jnp.cumsum(padded)
    offs = ends - padded
    sp1 = (offs[i1] + r1).astype(jnp.int32)
    sp2 = (offs[i2] + r2).astype(jnp.int32)
    tile_eid = jnp.searchsorted(
        ends, jnp.arange(N_TILES, dtype=jnp.int32) * TILE, side="right")
    tile_eid = jnp.minimum(tile_eid, E - 1).astype(jnp.int32)

    sp_all = jnp.concatenate(
        [sp1.reshape(NW, NCH, CHUNK), sp2.reshape(NW, NCH, CHUNK)], axis=1)
    g_all = jnp.concatenate(
        [g1.reshape(NW, NCH, CHUNK, 16), g2.reshape(NW, NCH, CHUNK, 16)],
        axis=1).reshape(NW, 2 * NCH * CHUNK, 16)

    x_sorted = _dispatch(x, sp_all)
    out_all = _grouped_mlp(tile_eid, x_sorted, W1, b1, W2, b2)
    return _combine(out_all, sp_all, g_all)


# speedup vs baseline: 3.7834x; 1.2227x over previous
import functools
import math

import jax
import jax.numpy as jnp
from jax import lax
from jax.experimental import pallas as pl
from jax.experimental.pallas import tpu as pltpu
from jax.experimental.pallas import tpu_sc as plsc

E = 8
K = 2
N = 2048
D = 768
H = 3072

TILE = 128
P = 5120
N_TILES = P // TILE
NC, NS = 2, 16
NW = NC * NS
TOK_W = N // NW

_SQRT2 = math.sqrt(2.0)


def _gelu_exact(v):
    return 0.5 * v * (1.0 + lax.erf(v / _SQRT2))


def _gating_body(x_ref, wg_ref, sp1_ref, sp2_ref, g1_ref, g2_ref,
                 te_ref, tv_ref, i1_s, i2_s, r1_s, r2_s):
    logits = jnp.dot(x_ref[...], wg_ref[...], preferred_element_type=jnp.float32)
    iota = lax.broadcasted_iota(jnp.int32, logits.shape, 1)
    m1 = jnp.max(logits, axis=1, keepdims=True)
    i1 = jnp.min(jnp.where(logits == m1, iota, E), axis=1, keepdims=True)
    masked = jnp.where(iota == i1, -jnp.inf, logits)
    m2 = jnp.max(masked, axis=1, keepdims=True)
    i2 = jnp.min(jnp.where(masked == m2, iota, E), axis=1, keepdims=True)
    a = jnp.exp(m2 - m1)
    lanes16 = jnp.ones((1, 16), jnp.float32)
    g1_ref[...] = (1.0 / (1.0 + a)) * lanes16
    g2_ref[...] = (a / (1.0 + a)) * lanes16
    i1_s[...] = i1
    i2_s[...] = i2

    ri = lax.broadcasted_iota(jnp.int32, (TILE, TILE), 0)
    ci = lax.broadcasted_iota(jnp.int32, (TILE, TILE), 1)
    l_strict = (ri > ci).astype(jnp.float32)
    e_blk = lax.broadcasted_iota(jnp.int32, (TILE, E), 1)

    def block(b, carry):
        oh1 = (i1_s[pl.ds(b * TILE, TILE), :] == e_blk).astype(jnp.float32)
        oh2 = (i2_s[pl.ds(b * TILE, TILE), :] == e_blk).astype(jnp.float32)
        oh = jnp.concatenate([oh1, oh2], axis=1)
        excl = jnp.dot(l_strict, oh, preferred_element_type=jnp.float32) + carry
        r1_s[pl.ds(b * TILE, TILE), :] = (
            jnp.sum(excl[:, :E] * oh1, axis=1, keepdims=True).astype(jnp.int32))
        r2_s[pl.ds(b * TILE, TILE), :] = (
            jnp.sum(excl[:, E:] * oh2, axis=1, keepdims=True).astype(jnp.int32))
        return carry + jnp.sum(oh, axis=0, keepdims=True)

    carry = lax.fori_loop(0, N // TILE, block,
                          jnp.zeros((1, 2 * E), jnp.float32))
    counts1 = carry[:, :E]
    counts = counts1 + carry[:, E:]

    padded = jnp.floor((counts + (TILE - 1)) * (1.0 / TILE)) * TILE
    ri8 = lax.broadcasted_iota(jnp.int32, (E, E), 0)
    ci8 = lax.broadcasted_iota(jnp.int32, (E, E), 1)
    u_incl = (ri8 <= ci8).astype(jnp.float32)
    ends = jnp.dot(padded, u_incl, preferred_element_type=jnp.float32)
    offs = ends - padded

    e_full = lax.broadcasted_iota(jnp.int32, (N, E), 1)
    oh1f = (i1_s[...] == e_full).astype(jnp.float32)
    oh2f = (i2_s[...] == e_full).astype(jnp.float32)
    sp1_ref[...] = (
        jnp.sum(oh1f * offs, axis=1, keepdims=True).astype(jnp.int32)
        + r1_s[...])
    sp2_ref[...] = (
        jnp.sum(oh2f * (offs + counts1), axis=1, keepdims=True).astype(jnp.int32)
        + r2_s[...])

    starts = (lax.broadcasted_iota(jnp.int32, (1, N_TILES), 1)
              * TILE).astype(jnp.float32)
    te = jnp.zeros((1, N_TILES), jnp.int32)
    for e in range(E - 1):
        te = te + (ends[:, e:e + 1] <= starts).astype(jnp.int32)
    te_ref[...] = te
    tv_ref[...] = (starts < ends[:, E - 1:E]).astype(jnp.int32)


def _gating(x, w_gate):
    sds = jax.ShapeDtypeStruct
    return pl.pallas_call(
        _gating_body,
        out_shape=(
            sds((N, 1), jnp.int32), sds((N, 1), jnp.int32),
            sds((N, 16), jnp.float32), sds((N, 16), jnp.float32),
            sds((1, N_TILES), jnp.int32), sds((1, N_TILES), jnp.int32),
        ),
        scratch_shapes=[
            pltpu.VMEM((N, 1), jnp.int32), pltpu.VMEM((N, 1), jnp.int32),
            pltpu.VMEM((N, 1), jnp.int32), pltpu.VMEM((N, 1), jnp.int32),
        ],
    )(x, w_gate)


def _dispatch_body(x_hbm, sp_hbm, xs_hbm, xbuf, idx_v):
    wid = lax.axis_index("s") * NC + lax.axis_index("c")
    pltpu.sync_copy(sp_hbm.at[wid], idx_v)
    pltpu.sync_copy(x_hbm.at[pl.ds(wid * TOK_W, TOK_W)], xbuf)
    pltpu.sync_copy(xbuf, xs_hbm.at[idx_v.at[0]])
    pltpu.sync_copy(xbuf, xs_hbm.at[idx_v.at[1]])


def _dispatch(x, sp_all):
    mesh = plsc.VectorSubcoreMesh(core_axis_name="c", subcore_axis_name="s",
                                  num_cores=NC, num_subcores=NS)
    return pl.kernel(
        _dispatch_body,
        out_type=jax.ShapeDtypeStruct((P, D), jnp.float32),
        mesh=mesh,
        scratch_types=[
            pltpu.VMEM((TOK_W, D), jnp.float32),
            pltpu.VMEM((2, TOK_W), jnp.int32),
        ],
    )(x, sp_all)


def _mlp_body(eid_ref, tv_ref, xs_ref, W1_ref, b1_ref, W2_ref, b2_ref, out_ref):
    i = pl.program_id(0)

    @pl.when(tv_ref[i] != 0)
    def _run():
        h = _gelu_exact(
            jnp.dot(xs_ref[...], W1_ref[0], preferred_element_type=jnp.float32)
            + b1_ref[0])
        out_ref[...] = (
            jnp.dot(h, W2_ref[0], preferred_element_type=jnp.float32)
            + b2_ref[0])


def _grouped_mlp(tile_eid, tile_valid, x_sorted, W1, b1, W2, b2):
    grid_spec = pltpu.PrefetchScalarGridSpec(
        num_scalar_prefetch=2,
        grid=(N_TILES,),
        in_specs=[
            pl.BlockSpec((TILE, D), lambda i, eid, tv: (i, 0)),
            pl.BlockSpec((1, D, H), lambda i, eid, tv: (eid[i], 0, 0)),
            pl.BlockSpec((1, 1, H), lambda i, eid, tv: (eid[i], 0, 0)),
            pl.BlockSpec((1, H, D), lambda i, eid, tv: (eid[i], 0, 0)),
            pl.BlockSpec((1, 1, D), lambda i, eid, tv: (eid[i], 0, 0)),
        ],
        out_specs=pl.BlockSpec((TILE, D), lambda i, eid, tv: (i, 0)),
    )
    return pl.pallas_call(
        _mlp_body,
        grid_spec=grid_spec,
        out_shape=jax.ShapeDtypeStruct((P, D), jnp.float32),
        compiler_params=pltpu.CompilerParams(
            dimension_semantics=("arbitrary",),
        ),
    )(tile_eid, tile_valid, x_sorted, W1,
      b1.reshape(E, 1, H), W2, b2.reshape(E, 1, D))


def _combine_body(out_hbm, sp_hbm, g_hbm, y_hbm, r1b, r2b, idx_v, g_v):
    wid = lax.axis_index("s") * NC + lax.axis_index("c")
    pltpu.sync_copy(sp_hbm.at[wid], idx_v)
    pltpu.sync_copy(g_hbm.at[wid], g_v)
    pltpu.sync_copy(out_hbm.at[idx_v.at[0]], r1b)
    pltpu.sync_copy(out_hbm.at[idx_v.at[1]], r2b)

    def token(j, _):
        ga = g_v[j, :]
        gb = g_v[TOK_W + j, :]
        for c in range(D // 16):
            av = r1b[j, pl.ds(c * 16, 16)]
            bv = r2b[j, pl.ds(c * 16, 16)]
            r1b[j, pl.ds(c * 16, 16)] = ga * av + gb * bv
        return 0

    lax.fori_loop(0, TOK_W, token, 0)
    pltpu.sync_copy(r1b, y_hbm.at[pl.ds(wid * TOK_W, TOK_W)])


def _combine(out_all, sp_all, g_all):
    mesh = plsc.VectorSubcoreMesh(core_axis_name="c", subcore_axis_name="s",
                                  num_cores=NC, num_subcores=NS)
    return pl.kernel(
        _combine_body,
        out_type=jax.ShapeDtypeStruct((N, D), jnp.float32),
        mesh=mesh,
        scratch_types=[
            pltpu.VMEM((TOK_W, D), jnp.float32),
            pltpu.VMEM((TOK_W, D), jnp.float32),
            pltpu.VMEM((2, TOK_W), jnp.int32),
            pltpu.VMEM((2 * TOK_W, 16), jnp.float32),
        ],
    )(out_all, sp_all, g_all)


@jax.jit
def kernel(x, w_gate, W1, b1, W2, b2):
    sp1, sp2, g1, g2, te, tv = _gating(x, w_gate)
    sp_all = jnp.stack(
        [sp1[:, 0].reshape(NW, TOK_W), sp2[:, 0].reshape(NW, TOK_W)], axis=1)
    g_all = jnp.concatenate(
        [g1.reshape(NW, TOK_W, 16), g2.reshape(NW, TOK_W, 16)], axis=1)

    x_sorted = _dispatch(x, sp_all)
    out_all = _grouped_mlp(te[0], tv[0], x_sorted, W1, b1, W2, b2)
    return _combine(out_all, sp_all, g_all)


# speedup vs baseline: 3.7963x; 1.0034x over previous
import functools
import math

import jax
import jax.numpy as jnp
from jax import lax
from jax.experimental import pallas as pl
from jax.experimental.pallas import tpu as pltpu
from jax.experimental.pallas import tpu_sc as plsc

E = 8
K = 2
N = 2048
D = 768
H = 3072

TILE = 128
P = 5120
N_TILES = P // TILE
NC, NS = 2, 16
NW = NC * NS
TOK_W = N // NW

_SQRT2 = math.sqrt(2.0)


def _gelu_exact(v):
    return 0.5 * v * (1.0 + lax.erf(v / _SQRT2))


def _gating_body(x_ref, wg_ref, sp1_ref, sp2_ref, g1_ref, g2_ref,
                 te_ref, tv_ref, i1_s, i2_s, r1_s, r2_s):
    logits = jnp.dot(x_ref[...], wg_ref[...], preferred_element_type=jnp.float32)
    iota = lax.broadcasted_iota(jnp.int32, logits.shape, 1)
    m1 = jnp.max(logits, axis=1, keepdims=True)
    i1 = jnp.min(jnp.where(logits == m1, iota, E), axis=1, keepdims=True)
    masked = jnp.where(iota == i1, -jnp.inf, logits)
    m2 = jnp.max(masked, axis=1, keepdims=True)
    i2 = jnp.min(jnp.where(masked == m2, iota, E), axis=1, keepdims=True)
    a = jnp.exp(m2 - m1)
    lanes16 = jnp.ones((1, 16), jnp.float32)
    g1_ref[...] = (1.0 / (1.0 + a)) * lanes16
    g2_ref[...] = (a / (1.0 + a)) * lanes16
    i1_s[...] = i1
    i2_s[...] = i2

    ri = lax.broadcasted_iota(jnp.int32, (TILE, TILE), 0)
    ci = lax.broadcasted_iota(jnp.int32, (TILE, TILE), 1)
    l_strict = (ri > ci).astype(jnp.float32)
    e_blk = lax.broadcasted_iota(jnp.int32, (TILE, E), 1)

    def block(b, carry):
        oh1 = (i1_s[pl.ds(b * TILE, TILE), :] == e_blk).astype(jnp.float32)
        oh2 = (i2_s[pl.ds(b * TILE, TILE), :] == e_blk).astype(jnp.float32)
        oh = jnp.concatenate([oh1, oh2], axis=1)
        excl = jnp.dot(l_strict, oh, preferred_element_type=jnp.float32) + carry
        r1_s[pl.ds(b * TILE, TILE), :] = (
            jnp.sum(excl[:, :E] * oh1, axis=1, keepdims=True).astype(jnp.int32))
        r2_s[pl.ds(b * TILE, TILE), :] = (
            jnp.sum(excl[:, E:] * oh2, axis=1, keepdims=True).astype(jnp.int32))
        return carry + jnp.sum(oh, axis=0, keepdims=True)

    carry = jnp.zeros((1, 2 * E), jnp.float32)
    for b in range(N // TILE):
        carry = block(b, carry)
    counts1 = carry[:, :E]
    counts = counts1 + carry[:, E:]

    padded = jnp.floor((counts + (TILE - 1)) * (1.0 / TILE)) * TILE
    ri8 = lax.broadcasted_iota(jnp.int32, (E, E), 0)
    ci8 = lax.broadcasted_iota(jnp.int32, (E, E), 1)
    u_incl = (ri8 <= ci8).astype(jnp.float32)
    ends = jnp.dot(padded, u_incl, preferred_element_type=jnp.float32)
    offs = ends - padded

    e_full = lax.broadcasted_iota(jnp.int32, (N, E), 1)
    oh1f = (i1_s[...] == e_full).astype(jnp.float32)
    oh2f = (i2_s[...] == e_full).astype(jnp.float32)
    sp1_ref[...] = (
        jnp.sum(oh1f * offs, axis=1, keepdims=True).astype(jnp.int32)
        + r1_s[...])
    sp2_ref[...] = (
        jnp.sum(oh2f * (offs + counts1), axis=1, keepdims=True).astype(jnp.int32)
        + r2_s[...])

    starts = (lax.broadcasted_iota(jnp.int32, (1, N_TILES), 1)
              * TILE).astype(jnp.float32)
    te = jnp.zeros((1, N_TILES), jnp.int32)
    for e in range(E - 1):
        te = te + (ends[:, e:e + 1] <= starts).astype(jnp.int32)
    te_ref[...] = te
    tv_ref[...] = (starts < ends[:, E - 1:E]).astype(jnp.int32)


def _gating(x, w_gate):
    sds = jax.ShapeDtypeStruct
    return pl.pallas_call(
        _gating_body,
        out_shape=(
            sds((N, 1), jnp.int32), sds((N, 1), jnp.int32),
            sds((N, 16), jnp.float32), sds((N, 16), jnp.float32),
            sds((1, N_TILES), jnp.int32), sds((1, N_TILES), jnp.int32),
        ),
        scratch_shapes=[
            pltpu.VMEM((N, 1), jnp.int32), pltpu.VMEM((N, 1), jnp.int32),
            pltpu.VMEM((N, 1), jnp.int32), pltpu.VMEM((N, 1), jnp.int32),
        ],
    )(x, w_gate)


def _dispatch_body(x_hbm, sp1_hbm, sp2_hbm, xs_hbm, xbuf, idx_v):
    wid = lax.axis_index("s") * NC + lax.axis_index("c")
    pltpu.sync_copy(sp1_hbm.at[wid], idx_v.at[0])
    pltpu.sync_copy(sp2_hbm.at[wid], idx_v.at[1])
    pltpu.sync_copy(x_hbm.at[pl.ds(wid * TOK_W, TOK_W)], xbuf)
    pltpu.sync_copy(xbuf, xs_hbm.at[idx_v.at[0]])
    pltpu.sync_copy(xbuf, xs_hbm.at[idx_v.at[1]])


def _dispatch(x, sp1, sp2):
    mesh = plsc.VectorSubcoreMesh(core_axis_name="c", subcore_axis_name="s",
                                  num_cores=NC, num_subcores=NS)
    return pl.kernel(
        _dispatch_body,
        out_type=jax.ShapeDtypeStruct((P, D), jnp.float32),
        mesh=mesh,
        scratch_types=[
            pltpu.VMEM((TOK_W, D), jnp.float32),
            pltpu.VMEM((2, TOK_W), jnp.int32),
        ],
    )(x, sp1, sp2)


def _mlp_body(eid_ref, tv_ref, xs_ref, W1_ref, b1_ref, W2_ref, b2_ref, out_ref):
    i = pl.program_id(0)

    @pl.when(tv_ref[i] != 0)
    def _run():
        h = _gelu_exact(
            jnp.dot(xs_ref[...], W1_ref[0], preferred_element_type=jnp.float32)
            + b1_ref[0])
        out_ref[...] = (
            jnp.dot(h, W2_ref[0], preferred_element_type=jnp.float32)
            + b2_ref[0])


def _grouped_mlp(tile_eid, tile_valid, x_sorted, W1, b1, W2, b2):
    grid_spec = pltpu.PrefetchScalarGridSpec(
        num_scalar_prefetch=2,
        grid=(N_TILES,),
        in_specs=[
            pl.BlockSpec((TILE, D), lambda i, eid, tv: (i, 0)),
            pl.BlockSpec((1, D, H), lambda i, eid, tv: (eid[i], 0, 0)),
            pl.BlockSpec((1, 1, H), lambda i, eid, tv: (eid[i], 0, 0)),
            pl.BlockSpec((1, H, D), lambda i, eid, tv: (eid[i], 0, 0)),
            pl.BlockSpec((1, 1, D), lambda i, eid, tv: (eid[i], 0, 0)),
        ],
        out_specs=pl.BlockSpec((TILE, D), lambda i, eid, tv: (i, 0)),
    )
    return pl.pallas_call(
        _mlp_body,
        grid_spec=grid_spec,
        out_shape=jax.ShapeDtypeStruct((P, D), jnp.float32),
        compiler_params=pltpu.CompilerParams(
            dimension_semantics=("arbitrary",),
        ),
    )(tile_eid, tile_valid, x_sorted, W1,
      b1.reshape(E, 1, H), W2, b2.reshape(E, 1, D))


def _combine_body(out_hbm, sp1_hbm, sp2_hbm, g1_hbm, g2_hbm, y_hbm,
                  r1b, r2b, idx_v, g_v):
    wid = lax.axis_index("s") * NC + lax.axis_index("c")
    pltpu.sync_copy(sp1_hbm.at[wid], idx_v.at[0])
    pltpu.sync_copy(sp2_hbm.at[wid], idx_v.at[1])
    pltpu.sync_copy(g1_hbm.at[wid], g_v.at[0])
    pltpu.sync_copy(g2_hbm.at[wid], g_v.at[1])
    pltpu.sync_copy(out_hbm.at[idx_v.at[0]], r1b)
    pltpu.sync_copy(out_hbm.at[idx_v.at[1]], r2b)

    def token(j, _):
        ga = g_v[0, j, :]
        gb = g_v[1, j, :]
        for c in range(D // 16):
            av = r1b[j, pl.ds(c * 16, 16)]
            bv = r2b[j, pl.ds(c * 16, 16)]
            r1b[j, pl.ds(c * 16, 16)] = ga * av + gb * bv
        return 0

    lax.fori_loop(0, TOK_W, token, 0)
    pltpu.sync_copy(r1b, y_hbm.at[pl.ds(wid * TOK_W, TOK_W)])


def _combine(out_all, sp1, sp2, g1, g2):
    mesh = plsc.VectorSubcoreMesh(core_axis_name="c", subcore_axis_name="s",
                                  num_cores=NC, num_subcores=NS)
    return pl.kernel(
        _combine_body,
        out_type=jax.ShapeDtypeStruct((N, D), jnp.float32),
        mesh=mesh,
        scratch_types=[
            pltpu.VMEM((TOK_W, D), jnp.float32),
            pltpu.VMEM((TOK_W, D), jnp.float32),
            pltpu.VMEM((2, TOK_W), jnp.int32),
            pltpu.VMEM((2, TOK_W, 16), jnp.float32),
        ],
    )(out_all, sp1, sp2, g1, g2)


@jax.jit
def kernel(x, w_gate, W1, b1, W2, b2):
    sp1, sp2, g1, g2, te, tv = _gating(x, w_gate)
    sp1 = sp1.reshape(NW, TOK_W)
    sp2 = sp2.reshape(NW, TOK_W)
    g1 = g1.reshape(NW, TOK_W, 16)
    g2 = g2.reshape(NW, TOK_W, 16)

    x_sorted = _dispatch(x, sp1, sp2)
    out_all = _grouped_mlp(te[0], tv[0], x_sorted, W1, b1, W2, b2)
    return _combine(out_all, sp1, sp2, g1, g2)


# speedup vs baseline: 3.9941x; 1.0521x over previous
import functools
import math

import jax
import jax.numpy as jnp
from jax import lax
from jax.experimental import pallas as pl
from jax.experimental.pallas import tpu as pltpu
from jax.experimental.pallas import tpu_sc as plsc

E = 8
K = 2
N = 2048
D = 768
H = 3072

TILE = 128
P = 5120
N_TILES = P // TILE
NC, NS = 2, 16
NW = NC * NS
TOK_W = N // NW

_SQRT2 = math.sqrt(2.0)


def _gelu_exact(v):
    return 0.5 * v * (1.0 + lax.erf(v / _SQRT2))


def _gating_body(x_ref, wg_ref, sp1_ref, sp2_ref, g1_ref, g2_ref,
                 te_ref, tv_ref, i1_s, i2_s, r1_s, r2_s):
    logits = jnp.dot(x_ref[...], wg_ref[...], preferred_element_type=jnp.float32)
    iota = lax.broadcasted_iota(jnp.int32, logits.shape, 1)
    m1 = jnp.max(logits, axis=1, keepdims=True)
    i1 = jnp.min(jnp.where(logits == m1, iota, E), axis=1, keepdims=True)
    masked = jnp.where(iota == i1, -jnp.inf, logits)
    m2 = jnp.max(masked, axis=1, keepdims=True)
    i2 = jnp.min(jnp.where(masked == m2, iota, E), axis=1, keepdims=True)
    a = jnp.exp(m2 - m1)
    lanes16 = jnp.ones((1, 16), jnp.float32)
    g1_ref[...] = (1.0 / (1.0 + a)) * lanes16
    g2_ref[...] = (a / (1.0 + a)) * lanes16
    i1_s[...] = i1
    i2_s[...] = i2

    ri = lax.broadcasted_iota(jnp.int32, (TILE, TILE), 0)
    ci = lax.broadcasted_iota(jnp.int32, (TILE, TILE), 1)
    l_strict = (ri > ci).astype(jnp.float32)
    e_blk = lax.broadcasted_iota(jnp.int32, (TILE, E), 1)

    def block(b, carry):
        oh1 = (i1_s[pl.ds(b * TILE, TILE), :] == e_blk).astype(jnp.float32)
        oh2 = (i2_s[pl.ds(b * TILE, TILE), :] == e_blk).astype(jnp.float32)
        oh = jnp.concatenate([oh1, oh2], axis=1)
        excl = jnp.dot(l_strict, oh, preferred_element_type=jnp.float32) + carry
        r1_s[pl.ds(b * TILE, TILE), :] = (
            jnp.sum(excl[:, :E] * oh1, axis=1, keepdims=True).astype(jnp.int32))
        r2_s[pl.ds(b * TILE, TILE), :] = (
            jnp.sum(excl[:, E:] * oh2, axis=1, keepdims=True).astype(jnp.int32))
        return carry + jnp.sum(oh, axis=0, keepdims=True)

    carry = jnp.zeros((1, 2 * E), jnp.float32)
    for b in range(N // TILE):
        carry = block(b, carry)
    counts1 = carry[:, :E]
    counts = counts1 + carry[:, E:]

    padded = jnp.floor((counts + (TILE - 1)) * (1.0 / TILE)) * TILE
    ri8 = lax.broadcasted_iota(jnp.int32, (E, E), 0)
    ci8 = lax.broadcasted_iota(jnp.int32, (E, E), 1)
    u_incl = (ri8 <= ci8).astype(jnp.float32)
    ends = jnp.dot(padded, u_incl, preferred_element_type=jnp.float32)
    offs = ends - padded

    e_full = lax.broadcasted_iota(jnp.int32, (N, E), 1)
    oh1f = (i1_s[...] == e_full).astype(jnp.float32)
    oh2f = (i2_s[...] == e_full).astype(jnp.float32)
    sp1_ref[...] = (
        jnp.sum(oh1f * offs, axis=1, keepdims=True).astype(jnp.int32)
        + r1_s[...])
    sp2_ref[...] = (
        jnp.sum(oh2f * (offs + counts1), axis=1, keepdims=True).astype(jnp.int32)
        + r2_s[...])

    starts = (lax.broadcasted_iota(jnp.int32, (1, N_TILES), 1)
              * TILE).astype(jnp.float32)
    te = jnp.zeros((1, N_TILES), jnp.int32)
    for e in range(E - 1):
        te = te + (ends[:, e:e + 1] <= starts).astype(jnp.int32)
    te_ref[...] = te
    tv_ref[...] = (starts < ends[:, E - 1:E]).astype(jnp.int32)


def _gating(x, w_gate):
    sds = jax.ShapeDtypeStruct
    return pl.pallas_call(
        _gating_body,
        out_shape=(
            sds((N, 1), jnp.int32), sds((N, 1), jnp.int32),
            sds((N, 16), jnp.float32), sds((N, 16), jnp.float32),
            sds((1, N_TILES), jnp.int32), sds((1, N_TILES), jnp.int32),
        ),
        scratch_shapes=[
            pltpu.VMEM((N, 1), jnp.int32), pltpu.VMEM((N, 1), jnp.int32),
            pltpu.VMEM((N, 1), jnp.int32), pltpu.VMEM((N, 1), jnp.int32),
        ],
    )(x, w_gate)


def _dispatch_body(x_hbm, sp1_hbm, sp2_hbm, xs_hbm, xbuf, idx_v):
    wid = lax.axis_index("s") * NC + lax.axis_index("c")
    pltpu.sync_copy(sp1_hbm.at[wid], idx_v.at[0])
    pltpu.sync_copy(sp2_hbm.at[wid], idx_v.at[1])
    pltpu.sync_copy(x_hbm.at[pl.ds(wid * TOK_W, TOK_W)], xbuf)
    pltpu.sync_copy(xbuf, xs_hbm.at[idx_v.at[0]])
    pltpu.sync_copy(xbuf, xs_hbm.at[idx_v.at[1]])


def _dispatch(x, sp1, sp2):
    mesh = plsc.VectorSubcoreMesh(core_axis_name="c", subcore_axis_name="s",
                                  num_cores=NC, num_subcores=NS)
    return pl.kernel(
        _dispatch_body,
        out_type=jax.ShapeDtypeStruct((P, D), jnp.float32),
        mesh=mesh,
        scratch_types=[
            pltpu.VMEM((TOK_W, D), jnp.float32),
            pltpu.VMEM((2, TOK_W), jnp.int32),
        ],
    )(x, sp1, sp2)


def _mlp_body(te_ref, tv_ref, par_ref, chg_ref, nx_ref, hn_ref,
              xs_ref, W1_hbm, b1_ref, W2_hbm, b2_ref, out_ref,
              w1b, w2b, s1, s2):
    i = pl.program_id(0)
    e = te_ref[i]
    par = par_ref[i]

    @pl.when(i == 0)
    def _first():
        c1 = pltpu.make_async_copy(W1_hbm.at[e], w1b.at[0], s1)
        c2 = pltpu.make_async_copy(W2_hbm.at[e], w2b.at[0], s2)
        c1.start()
        c2.start()
        c1.wait()
        c2.wait()

    @pl.when((i > 0) & (chg_ref[i] == 1))
    def _arrive():
        pltpu.make_async_copy(W1_hbm.at[e], w1b.at[par], s1).wait()
        pltpu.make_async_copy(W2_hbm.at[e], w2b.at[par], s2).wait()

    @pl.when(((i == 0) | (chg_ref[i] == 1)) & (hn_ref[i] == 1))
    def _launch_next():
        nx = nx_ref[i]
        pltpu.make_async_copy(W1_hbm.at[nx], w1b.at[1 - par], s1).start()
        pltpu.make_async_copy(W2_hbm.at[nx], w2b.at[1 - par], s2).start()

    @pl.when(tv_ref[i] != 0)
    def _run():
        h = _gelu_exact(
            jnp.dot(xs_ref[...], w1b[par], preferred_element_type=jnp.float32)
            + b1_ref[0])
        out_ref[...] = (
            jnp.dot(h, w2b[par], preferred_element_type=jnp.float32)
            + b2_ref[0])


def _grouped_mlp(te, tv, par, chg, nx, hn, x_sorted, W1, b1, W2, b2):
    hbm = pl.BlockSpec(memory_space=pltpu.MemorySpace.HBM)
    grid_spec = pltpu.PrefetchScalarGridSpec(
        num_scalar_prefetch=6,
        grid=(N_TILES,),
        in_specs=[
            pl.BlockSpec((TILE, D), lambda i, *s: (i, 0)),
            hbm,
            pl.BlockSpec((1, 1, H), lambda i, te, *s: (te[i], 0, 0)),
            hbm,
            pl.BlockSpec((1, 1, D), lambda i, te, *s: (te[i], 0, 0)),
        ],
        out_specs=pl.BlockSpec((TILE, D), lambda i, *s: (i, 0)),
        scratch_shapes=[
            pltpu.VMEM((2, D, H), jnp.float32),
            pltpu.VMEM((2, H, D), jnp.float32),
            pltpu.SemaphoreType.DMA,
            pltpu.SemaphoreType.DMA,
        ],
    )
    return pl.pallas_call(
        _mlp_body,
        grid_spec=grid_spec,
        out_shape=jax.ShapeDtypeStruct((P, D), jnp.float32),
        compiler_params=pltpu.CompilerParams(
            dimension_semantics=("arbitrary",),
        ),
    )(te, tv, par, chg, nx, hn, x_sorted, W1,
      b1.reshape(E, 1, H), W2, b2.reshape(E, 1, D))


def _combine_body(out_hbm, sp1_hbm, sp2_hbm, g1_hbm, g2_hbm, y_hbm,
                  r1b, r2b, idx_v, g_v):
    wid = lax.axis_index("s") * NC + lax.axis_index("c")
    pltpu.sync_copy(sp1_hbm.at[wid], idx_v.at[0])
    pltpu.sync_copy(sp2_hbm.at[wid], idx_v.at[1])
    pltpu.sync_copy(g1_hbm.at[wid], g_v.at[0])
    pltpu.sync_copy(g2_hbm.at[wid], g_v.at[1])
    pltpu.sync_copy(out_hbm.at[idx_v.at[0]], r1b)
    pltpu.sync_copy(out_hbm.at[idx_v.at[1]], r2b)

    def token(j, _):
        ga = g_v[0, j, :]
        gb = g_v[1, j, :]
        for c in range(D // 16):
            av = r1b[j, pl.ds(c * 16, 16)]
            bv = r2b[j, pl.ds(c * 16, 16)]
            r1b[j, pl.ds(c * 16, 16)] = ga * av + gb * bv
        return 0

    lax.fori_loop(0, TOK_W, token, 0)
    pltpu.sync_copy(r1b, y_hbm.at[pl.ds(wid * TOK_W, TOK_W)])


def _combine(out_all, sp1, sp2, g1, g2):
    mesh = plsc.VectorSubcoreMesh(core_axis_name="c", subcore_axis_name="s",
                                  num_cores=NC, num_subcores=NS)
    return pl.kernel(
        _combine_body,
        out_type=jax.ShapeDtypeStruct((N, D), jnp.float32),
        mesh=mesh,
        scratch_types=[
            pltpu.VMEM((TOK_W, D), jnp.float32),
            pltpu.VMEM((TOK_W, D), jnp.float32),
            pltpu.VMEM((2, TOK_W), jnp.int32),
            pltpu.VMEM((2, TOK_W, 16), jnp.float32),
        ],
    )(out_all, sp1, sp2, g1, g2)


@jax.jit
def kernel(x, w_gate, W1, b1, W2, b2):
    sp1, sp2, g1, g2, te, tv = _gating(x, w_gate)
    sp1 = sp1.reshape(NW, TOK_W)
    sp2 = sp2.reshape(NW, TOK_W)
    g1 = g1.reshape(NW, TOK_W, 16)
    g2 = g2.reshape(NW, TOK_W, 16)

    te_a, tv_a = te[0], tv[0]
    chg = jnp.concatenate(
        [jnp.zeros((1,), jnp.int32), (te_a[1:] != te_a[:-1]).astype(jnp.int32)])
    par = (jnp.cumsum(chg) % 2).astype(jnp.int32)
    idx = jnp.arange(N_TILES, dtype=jnp.int32)
    big = jnp.where(chg == 1, idx, N_TILES + 1)
    sufmin = lax.associative_scan(jnp.minimum, big, reverse=True)
    nxtb = jnp.concatenate([sufmin[1:], jnp.full((1,), N_TILES + 1, jnp.int32)])
    hn = (nxtb <= N_TILES).astype(jnp.int32)
    nx = te_a[jnp.clip(nxtb, 0, N_TILES - 1)]

    x_sorted = _dispatch(x, sp1, sp2)
    out_all = _grouped_mlp(te_a, tv_a, par, chg, nx, hn,
                           x_sorted, W1, b1, W2, b2)
    return _combine(out_all, sp1, sp2, g1, g2)
